# baseline (speedup 1.0000x reference)
import functools

import jax
import jax.numpy as jnp
from jax import lax
from jax.experimental import pallas as pl
from jax.experimental.pallas import tpu as pltpu
from jax.experimental.pallas import tpu_sc as plsc

_NC = 2
_NS = 16
_NW = _NC * _NS
_GRP = 128


_NB = 4


def _make_gather(n_rows: int, d: int):
  assert n_rows % (_NW * _GRP) == 0
  g_per_w = n_rows // (_NW * _GRP)
  assert g_per_w % _NB == 0
  mesh = plsc.VectorSubcoreMesh(core_axis_name="c", subcore_axis_name="s")

  @functools.partial(
      pl.kernel,
      out_type=jax.ShapeDtypeStruct((n_rows, d), jnp.float32),
      mesh=mesh,
      scratch_types=[
          pltpu.VMEM((g_per_w, _GRP), jnp.int32),
          pltpu.VMEM((_NB, _GRP, d), jnp.float32),
          pltpu.SemaphoreType.DMA((_NB,)),
          pltpu.SemaphoreType.DMA((_NB,)),
      ],
      compiler_params=pltpu.CompilerParams(
          use_tc_tiling_on_sc=False, skip_device_barrier=True),
  )
  def gather_kernel(table_hbm, idx_hbm, out_hbm, idx_v, rows_v, in_sem,
                    out_sem):
    wid = lax.axis_index("s") * _NC + lax.axis_index("c")
    gbase = wid * g_per_w
    pltpu.sync_copy(idx_hbm.at[pl.ds(gbase, g_per_w)], idx_v)

    def out_slice(t):
      return out_hbm.at[pl.ds((gbase + t) * _GRP, _GRP)]

    for b in range(_NB):
      pltpu.async_copy(table_hbm.at[idx_v.at[b]], rows_v.at[b], in_sem.at[b])

    @pl.loop(0, g_per_w, step=_NB)
    def _outer(g):
      for b in range(_NB):
        t = g + b
        pltpu.make_async_copy(table_hbm.at[idx_v.at[t]], rows_v.at[b],
                              in_sem.at[b]).wait()
        pltpu.async_copy(rows_v.at[b], out_slice(t), out_sem.at[b])

        @pl.when(t + _NB < g_per_w)
        def _():
          pltpu.make_async_copy(rows_v.at[b], out_slice(t),
                                out_sem.at[b]).wait()
          pltpu.async_copy(table_hbm.at[idx_v.at[t + _NB]], rows_v.at[b],
                           in_sem.at[b])

    for b in range(_NB):
      t = g_per_w - _NB + b
      pltpu.make_async_copy(rows_v.at[b], out_slice(t), out_sem.at[b]).wait()

  return gather_kernel


def kernel(x, mask_ids, table):
  del mask_ids
  b, l = x.shape
  _, d = table.shape
  n = b * l
  idx = x.reshape(n // _GRP, _GRP).astype(jnp.int32)
  out = _make_gather(n, d)(table, idx)
  return out.reshape(b, l, d), jnp.asarray(0.0, dtype=jnp.float32)


# speedup vs baseline: 1.2214x; 1.2214x over previous
import functools

import jax
import jax.numpy as jnp
from jax import lax
from jax.experimental import pallas as pl
from jax.experimental.pallas import tpu as pltpu
from jax.experimental.pallas import tpu_sc as plsc

_NC = 2
_NS = 16
_NW = _NC * _NS
_GRP = 128
_NB = 2


def _make_gather(n_rows: int, d: int):
  assert n_rows % (_NW * _GRP) == 0
  g_per_w = n_rows // (_NW * _GRP)
  assert g_per_w % _NB == 0
  mesh = plsc.VectorSubcoreMesh(core_axis_name="c", subcore_axis_name="s")

  @functools.partial(
      pl.kernel,
      out_type=jax.ShapeDtypeStruct((n_rows, d), jnp.float32),
      mesh=mesh,
      scratch_types=[
          pltpu.VMEM((g_per_w, _GRP), jnp.int32),
          pltpu.VMEM((_NB, _GRP, 2 * d), jnp.float32),
          pltpu.VMEM((_NB, _GRP, d), jnp.float32),
          pltpu.SemaphoreType.DMA((_NB,)),
          pltpu.SemaphoreType.DMA((_NB,)),
      ],
      compiler_params=pltpu.CompilerParams(use_tc_tiling_on_sc=True),
  )
  def gather_kernel(t2_hbm, idx_hbm, out_hbm, idx_v, gbuf, pbuf, in_sem,
                    out_sem):
    wid = lax.axis_index("s") * _NC + lax.axis_index("c")
    gbase = wid * g_per_w
    pltpu.sync_copy(idx_hbm.at[pl.ds(gbase, g_per_w)], idx_v)

    def out_slice(t):
      return out_hbm.at[pl.ds((gbase + t) * _GRP, _GRP)]

    def bridge(b):
      @pl.loop(0, _GRP)
      def _row(i):
        for c in range(d // 16):
          pbuf[b, i, pl.ds(c * 16, 16)] = gbuf[b, i, pl.ds(c * 16, 16)]

    for b in range(_NB):
      pltpu.async_copy(t2_hbm.at[idx_v.at[b]], gbuf.at[b], in_sem.at[b])

    @pl.loop(0, g_per_w, step=_NB)
    def _outer(g):
      for b in range(_NB):
        t = g + b
        pltpu.make_async_copy(t2_hbm.at[idx_v.at[t]], gbuf.at[b],
                              in_sem.at[b]).wait()
        @pl.when(t >= _NB)
        def _():
          pltpu.make_async_copy(pbuf.at[b], out_slice(t - _NB),
                                out_sem.at[b]).wait()

        bridge(b)
        pltpu.async_copy(pbuf.at[b], out_slice(t), out_sem.at[b])

        @pl.when(t + _NB < g_per_w)
        def _():
          pltpu.async_copy(t2_hbm.at[idx_v.at[t + _NB]], gbuf.at[b],
                           in_sem.at[b])

    for b in range(_NB):
      t = g_per_w - _NB + b
      pltpu.make_async_copy(pbuf.at[b], out_slice(t), out_sem.at[b]).wait()

  return gather_kernel


def kernel(x, mask_ids, table):
  del mask_ids
  b, l = x.shape
  _, d = table.shape
  n = b * l
  idx = x.reshape(n // _GRP, _GRP).astype(jnp.int32)
  t2 = jnp.pad(table, ((0, 0), (0, d)))
  out = _make_gather(n, d)(t2, idx)
  return out.reshape(b, l, d), jnp.asarray(0.0, dtype=jnp.float32)
